# baseline (speedup 1.0000x reference)
import functools

import jax
import jax.numpy as jnp
import numpy as np
from jax import lax
from jax.experimental import pallas as pl
from jax.experimental.pallas import tpu as pltpu
from jax.experimental.pallas import tpu_sc as plsc

N_NODES = 10000
N_EDGES = 320000
FEAT = 128

_NC, _NS, _L = 2, 16, 16
_NW = _NC * _NS
_CH = 128
_NCHUNKS = N_EDGES // _CH


_GCH = 16
_GE = _GCH * _CH
_NG = 6
_STRIDE = 79


def _sc_scatter_body(src_hbm, dst_hbm, a_hbm, *scr):
    srcA, dstA, srcB, dstB = scr[0:4]
    idx1, idx2, ones_v, semA, semB, semS = scr[4:]

    wid = lax.axis_index("s") * _NC + lax.axis_index("c")
    for v in range(_GE // _L):
        ones_v[pl.ds(v * _L, _L)] = jnp.full((_L,), 1.0, jnp.float32)

    start = wid * _STRIDE * _CH
    max_base = N_EDGES - _GE

    def issue_loads(g, sbuf, dbuf, sem):
        base = jnp.minimum(start + g * _GE, max_base)
        pltpu.async_copy(src_hbm.at[pl.ds(base, _GE)], sbuf, sem)
        pltpu.async_copy(dst_hbm.at[pl.ds(base, _GE)], dbuf, sem)

    def drain_loads(sbuf, dbuf, sem):
        pltpu.make_async_copy(src_hbm.at[pl.ds(0, _GE)], sbuf, sem).wait()
        pltpu.make_async_copy(dst_hbm.at[pl.ds(0, _GE)], dbuf, sem).wait()

    def process_group(sbuf, dbuf):
        for v in range(_GE // _L):
            sl = pl.ds(v * _L, _L)
            sv = sbuf[sl]
            dv = dbuf[sl]
            idx1[sl] = sv * N_NODES + dv
            idx2[sl] = dv * N_NODES + sv
        return [pltpu.async_copy(ones_v, a_hbm.at[idx1], semS),
                pltpu.async_copy(ones_v, a_hbm.at[idx2], semS)]

    issue_loads(0, srcA, dstA, semA)
    issue_loads(1, srcB, dstB, semB)

    def body(g2, carry):
        ga = 2 * g2
        drain_loads(srcA, dstA, semA)
        cps = process_group(srcA, dstA)
        issue_loads(ga + 2, srcA, dstA, semA)
        for cp in cps:
            cp.wait()
        gb = ga + 1
        drain_loads(srcB, dstB, semB)
        cps = process_group(srcB, dstB)
        issue_loads(gb + 2, srcB, dstB, semB)
        for cp in cps:
            cp.wait()
        return carry

    lax.fori_loop(0, _NG // 2, body, 0)
    drain_loads(srcA, dstA, semA)
    drain_loads(srcB, dstB, semB)


@functools.cache
def _get_sc_scatter():
    scratch = [
        pltpu.VMEM((_GE,), jnp.int32),
        pltpu.VMEM((_GE,), jnp.int32),
        pltpu.VMEM((_GE,), jnp.int32),
        pltpu.VMEM((_GE,), jnp.int32),
    ]
    scratch += [
        pltpu.VMEM((_GE,), jnp.int32),
        pltpu.VMEM((_GE,), jnp.int32),
        pltpu.VMEM((_GE,), jnp.float32),
        pltpu.SemaphoreType.DMA,
        pltpu.SemaphoreType.DMA,
        pltpu.SemaphoreType.DMA,
    ]
    return pl.kernel(
        _sc_scatter_body,
        out_type=(),
        mesh=plsc.VectorSubcoreMesh(core_axis_name="c", subcore_axis_name="s"),
        scratch_types=scratch,
    )

_BM = 400
_MI = N_NODES // _BM


def _tc_body(a_blk, xaug_blk, x_blk, w1_ref, b1_ref, w2_ref, b2_ref,
             w3_ref, b3_ref, upd_ref, tap_ref, ssum_ref, viol_ref):
    i = pl.program_id(0)

    a16 = a_blk[...].astype(jnp.bfloat16)
    acc = jnp.dot(a16, xaug_blk[...],
                  preferred_element_type=jnp.float32)

    x = x_blk[...]
    nsum = acc[:, :FEAT]
    cnt = acc[:, FEAT:FEAT + 1]
    has_nb = cnt > 0.0
    nmean = nsum / jnp.maximum(cnt, 1.0)
    combined = jnp.concatenate([x, nmean], axis=1)
    h = jnp.maximum(
        jnp.dot(combined, w1_ref[...],
                preferred_element_type=jnp.float32) + b1_ref[...], 0.0)
    h = jnp.maximum(
        jnp.dot(h, w2_ref[...],
                preferred_element_type=jnp.float32) + b2_ref[...], 0.0)
    logits = jnp.sum(h * w3_ref[...], axis=1, keepdims=True) + b3_ref[...]
    score = jax.nn.sigmoid(logits)
    gain = jnp.where(has_nb, 0.05 * score, 0.0)
    upd_ref[...] = x + gain * jnp.tanh(x)
    tap = jnp.where(has_nb, score, 1.0)
    tap_ref[...] = tap
    part_sum = jnp.sum(tap).reshape(1, 1)
    part_viol = jnp.sum((tap < 0.7).astype(jnp.int32)).reshape(1, 1)

    @pl.when(i == 0)
    def _first():
        ssum_ref[...] = part_sum
        viol_ref[...] = part_viol

    @pl.when(i > 0)
    def _rest():
        ssum_ref[...] += part_sum
        viol_ref[...] += part_viol


_tc_fused = pl.pallas_call(
    _tc_body,
    grid=(_MI,),
    in_specs=[
        pl.BlockSpec((_BM, N_NODES), lambda i: (i, 0)),
        pl.BlockSpec((N_NODES, 2 * FEAT), lambda i: (0, 0)),
        pl.BlockSpec((_BM, FEAT), lambda i: (i, 0)),
        pl.BlockSpec((2 * FEAT, 64), lambda i: (0, 0)),
        pl.BlockSpec((1, 64), lambda i: (0, 0)),
        pl.BlockSpec((64, 32), lambda i: (0, 0)),
        pl.BlockSpec((1, 32), lambda i: (0, 0)),
        pl.BlockSpec((1, 32), lambda i: (0, 0)),
        pl.BlockSpec((1, 1), lambda i: (0, 0)),
    ],
    out_specs=[
        pl.BlockSpec((_BM, FEAT), lambda i: (i, 0)),
        pl.BlockSpec((_BM, 1), lambda i: (i, 0)),
        pl.BlockSpec((1, 1), lambda i: (0, 0)),
        pl.BlockSpec((1, 1), lambda i: (0, 0)),
    ],
    out_shape=[
        jax.ShapeDtypeStruct((N_NODES, FEAT), jnp.float32),
        jax.ShapeDtypeStruct((N_NODES, 1), jnp.float32),
        jax.ShapeDtypeStruct((1, 1), jnp.float32),
        jax.ShapeDtypeStruct((1, 1), jnp.int32),
    ],
    compiler_params=pltpu.CompilerParams(
        dimension_semantics=("arbitrary",)),
)


def kernel(node_features, edge_index, node_positions, node_radii,
           W1, b1, W2, b2, W3, b3):
    del node_positions, node_radii
    src = edge_index[0].astype(jnp.int32)
    dst = edge_index[1].astype(jnp.int32)

    a_ref = jax.new_ref(jnp.zeros((N_NODES * N_NODES,), jnp.float32))
    _get_sc_scatter()(src, dst, a_ref)
    A = a_ref[...].reshape(N_NODES, N_NODES)

    xaug = jnp.concatenate(
        [node_features,
         jnp.ones((N_NODES, 1), jnp.float32),
         jnp.zeros((N_NODES, FEAT - 1), jnp.float32)], axis=1
    ).astype(jnp.bfloat16)

    updated, tap, ssum, viol = _tc_fused(
        A, xaug, node_features,
        W1, b1.reshape(1, 64), W2, b2.reshape(1, 32),
        W3.reshape(1, 32), b3.reshape(1, 1))

    tapering_scores = tap[:, 0]
    avg_consistency = ssum[0, 0] / np.float32(N_NODES)
    num_violations = viol[0, 0]
    return updated, tapering_scores, avg_consistency, num_violations


# speedup vs baseline: 1.3040x; 1.3040x over previous
import functools

import jax
import jax.numpy as jnp
import numpy as np
from jax import lax
from jax.experimental import pallas as pl
from jax.experimental.pallas import tpu as pltpu
from jax.experimental.pallas import tpu_sc as plsc

N_NODES = 10000
N_PAD = 10240
N_EDGES = 320000
FEAT = 128

_NC, _NS, _L = 2, 16, 16
_NW = _NC * _NS
_CH = 128
_NCHUNKS = N_EDGES // _CH

_GCH = 16
_GE = _GCH * _CH
_NG = 5
_STRIDE = 79


def _sc_scatter_body(src_hbm, dst_hbm, a_hbm, *scr):
    srcA, dstA, srcB, dstB, idxc, ones_v, semA, semB, semS = scr

    wid = lax.axis_index("s") * _NC + lax.axis_index("c")
    for v in range(_GE // _L):
        ones_v[pl.ds(v * _L, _L)] = jnp.full((_L,), 1.0, jnp.float32)

    start = wid * _STRIDE * _CH
    max_base = N_EDGES - _GE

    def issue_loads(g, sbuf, dbuf, sem):
        base = jnp.minimum(start + g * _GE, max_base)
        pltpu.async_copy(src_hbm.at[pl.ds(base, _GE)], sbuf, sem)
        pltpu.async_copy(dst_hbm.at[pl.ds(base, _GE)], dbuf, sem)

    def drain_loads(sbuf, dbuf, sem):
        pltpu.make_async_copy(src_hbm.at[pl.ds(0, _GE)], sbuf, sem).wait()
        pltpu.make_async_copy(dst_hbm.at[pl.ds(0, _GE)], dbuf, sem).wait()

    def process_group(sbuf, dbuf):
        for v in range(_GE // _L):
            sl = pl.ds(v * _L, _L)
            sv = sbuf[sl]
            dv = dbuf[sl]
            lo = jnp.minimum(sv, dv)
            hi = jnp.maximum(sv, dv)
            idxc[sl] = lo * N_PAD + hi
        return [pltpu.async_copy(ones_v, a_hbm.at[idxc], semS)]

    issue_loads(0, srcA, dstA, semA)
    issue_loads(1, srcB, dstB, semB)

    def body(g2, carry):
        ga = 2 * g2
        drain_loads(srcA, dstA, semA)
        cps = process_group(srcA, dstA)
        issue_loads(ga + 2, srcA, dstA, semA)
        for cp in cps:
            cp.wait()
        gb = ga + 1
        drain_loads(srcB, dstB, semB)
        cps = process_group(srcB, dstB)
        issue_loads(gb + 2, srcB, dstB, semB)
        for cp in cps:
            cp.wait()
        return carry

    lax.fori_loop(0, _NG // 2, body, 0)
    drain_loads(srcA, dstA, semA)
    cps = process_group(srcA, dstA)
    for cp in cps:
        cp.wait()
    drain_loads(srcB, dstB, semB)


@functools.cache
def _get_sc_scatter():
    scratch = [
        pltpu.VMEM((_GE,), jnp.int32),
        pltpu.VMEM((_GE,), jnp.int32),
        pltpu.VMEM((_GE,), jnp.int32),
        pltpu.VMEM((_GE,), jnp.int32),
        pltpu.VMEM((_GE,), jnp.int32),
        pltpu.VMEM((_GE,), jnp.float32),
        pltpu.SemaphoreType.DMA,
        pltpu.SemaphoreType.DMA,
        pltpu.SemaphoreType.DMA,
    ]
    return pl.kernel(
        _sc_scatter_body,
        out_type=(),
        mesh=plsc.VectorSubcoreMesh(core_axis_name="c", subcore_axis_name="s"),
        scratch_types=scratch,
    )


_BM = 512
_NB = N_PAD // _BM
_NJ = _NB // 2 + 1


def _acc_body(u_blk, xaug_ref, acc_out, acc_ref):
    i = pl.program_id(0)
    jj = pl.program_id(1)
    j = lax.rem(i + jj, _NB)
    r = jnp.minimum(i, j)
    c = jnp.maximum(i, j)

    @pl.when((i == 0) & (jj == 0))
    def _init():
        acc_ref[...] = jnp.zeros_like(acc_ref)

    @pl.when((jj < _NJ - 1) | (i < _NB // 2))
    def _accumulate():
        blk = u_blk[...].astype(jnp.bfloat16)
        x_c = xaug_ref[pl.ds(c * _BM, _BM), :]
        x_r = xaug_ref[pl.ds(r * _BM, _BM), :]
        acc_ref[pl.ds(r * _BM, _BM), :] += jnp.dot(
            blk, x_c, preferred_element_type=jnp.float32)
        ir = lax.broadcasted_iota(jnp.int32, (_BM, _BM), 0)
        ic = lax.broadcasted_iota(jnp.int32, (_BM, _BM), 1)
        tblk = jnp.where((r == c) & (ir == ic), jnp.bfloat16(0.0), blk)
        acc_ref[pl.ds(c * _BM, _BM), :] += lax.dot_general(
            tblk, x_r, (((0,), (0,)), ((), ())),
            preferred_element_type=jnp.float32)

    @pl.when((i == _NB - 1) & (jj == _NJ - 1))
    def _flush():
        acc_out[...] = acc_ref[...]


_tc_acc = pl.pallas_call(
    _acc_body,
    grid=(_NB, _NJ),
    in_specs=[
        pl.BlockSpec((_BM, _BM),
                     lambda i, jj: (jnp.minimum(i, lax.rem(i + jj, _NB)),
                                    jnp.maximum(i, lax.rem(i + jj, _NB)))),
        pl.BlockSpec((N_PAD, 2 * FEAT), lambda i, jj: (0, 0)),
    ],
    out_specs=pl.BlockSpec((N_PAD, 2 * FEAT), lambda i, jj: (0, 0)),
    out_shape=jax.ShapeDtypeStruct((N_PAD, 2 * FEAT), jnp.float32),
    scratch_shapes=[pltpu.VMEM((N_PAD, 2 * FEAT), jnp.float32)],
    compiler_params=pltpu.CompilerParams(
        dimension_semantics=("arbitrary", "arbitrary")),
)


def _mlp_body(acc_blk, x_blk, w1_ref, b1_ref, w2_ref, b2_ref,
              w3_ref, b3_ref, upd_ref, tap_ref, ssum_ref, viol_ref):
    i = pl.program_id(0)

    x = x_blk[...]
    acc = acc_blk[...]
    nsum = acc[:, :FEAT]
    cnt = acc[:, FEAT:FEAT + 1]
    has_nb = cnt > 0.0
    nmean = nsum / jnp.maximum(cnt, 1.0)
    combined = jnp.concatenate([x, nmean], axis=1)
    h = jnp.maximum(
        jnp.dot(combined, w1_ref[...],
                preferred_element_type=jnp.float32) + b1_ref[...], 0.0)
    h = jnp.maximum(
        jnp.dot(h, w2_ref[...],
                preferred_element_type=jnp.float32) + b2_ref[...], 0.0)
    logits = jnp.sum(h * w3_ref[...], axis=1, keepdims=True) + b3_ref[...]
    score = jax.nn.sigmoid(logits)
    gain = jnp.where(has_nb, 0.05 * score, 0.0)
    upd_ref[...] = x + gain * jnp.tanh(x)
    tap = jnp.where(has_nb, score, 1.0)
    tap_ref[...] = tap
    part_sum = jnp.sum(tap).reshape(1, 1)
    part_viol = jnp.sum((tap < 0.7).astype(jnp.int32)).reshape(1, 1)

    @pl.when(i == 0)
    def _first():
        ssum_ref[...] = part_sum
        viol_ref[...] = part_viol

    @pl.when(i > 0)
    def _rest():
        ssum_ref[...] += part_sum
        viol_ref[...] += part_viol


_tc_mlp = pl.pallas_call(
    _mlp_body,
    grid=(_NB,),
    in_specs=[
        pl.BlockSpec((_BM, 2 * FEAT), lambda i: (i, 0)),
        pl.BlockSpec((_BM, FEAT), lambda i: (i, 0)),
        pl.BlockSpec((2 * FEAT, 64), lambda i: (0, 0)),
        pl.BlockSpec((1, 64), lambda i: (0, 0)),
        pl.BlockSpec((64, 32), lambda i: (0, 0)),
        pl.BlockSpec((1, 32), lambda i: (0, 0)),
        pl.BlockSpec((1, 32), lambda i: (0, 0)),
        pl.BlockSpec((1, 1), lambda i: (0, 0)),
    ],
    out_specs=[
        pl.BlockSpec((_BM, FEAT), lambda i: (i, 0)),
        pl.BlockSpec((_BM, 1), lambda i: (i, 0)),
        pl.BlockSpec((1, 1), lambda i: (0, 0)),
        pl.BlockSpec((1, 1), lambda i: (0, 0)),
    ],
    out_shape=[
        jax.ShapeDtypeStruct((N_PAD, FEAT), jnp.float32),
        jax.ShapeDtypeStruct((N_PAD, 1), jnp.float32),
        jax.ShapeDtypeStruct((1, 1), jnp.float32),
        jax.ShapeDtypeStruct((1, 1), jnp.int32),
    ],
    compiler_params=pltpu.CompilerParams(
        dimension_semantics=("arbitrary",)),
)


def kernel(node_features, edge_index, node_positions, node_radii,
           W1, b1, W2, b2, W3, b3):
    del node_positions, node_radii
    src = edge_index[0].astype(jnp.int32)
    dst = edge_index[1].astype(jnp.int32)

    a_ref = jax.new_ref(jnp.zeros((N_PAD * N_PAD,), jnp.float32))
    _get_sc_scatter()(src, dst, a_ref)
    U = a_ref[...].reshape(N_PAD, N_PAD)

    xaug = jnp.concatenate(
        [node_features,
         jnp.ones((N_NODES, 1), jnp.float32),
         jnp.zeros((N_NODES, FEAT - 1), jnp.float32)], axis=1
    ).astype(jnp.bfloat16)
    xaug_p = jnp.pad(xaug, ((0, N_PAD - N_NODES), (0, 0)))
    x_p = jnp.pad(node_features, ((0, N_PAD - N_NODES), (0, 0)))

    acc = _tc_acc(U, xaug_p)
    updated_p, tap_p, ssum, viol = _tc_mlp(
        acc, x_p,
        W1, b1.reshape(1, 64), W2, b2.reshape(1, 32),
        W3.reshape(1, 32), b3.reshape(1, 1))

    updated = updated_p[:N_NODES]
    tapering_scores = tap_p[:N_NODES, 0]
    avg_consistency = (ssum[0, 0] - np.float32(N_PAD - N_NODES)) / np.float32(N_NODES)
    num_violations = viol[0, 0]
    return updated, tapering_scores, avg_consistency, num_violations


# speedup vs baseline: 1.4168x; 1.0865x over previous
import functools

import jax
import jax.numpy as jnp
import numpy as np
from jax import lax
from jax.experimental import pallas as pl
from jax.experimental.pallas import tpu as pltpu
from jax.experimental.pallas import tpu_sc as plsc

N_NODES = 10000
N_PAD = 10240
N_EDGES = 320000
FEAT = 128

_NC, _NS, _L = 2, 16, 16
_NW = _NC * _NS
_CH = 128
_NCHUNKS = N_EDGES // _CH

_GCH = 16
_GE = _GCH * _CH
_NG = 5
_STRIDE = 79


def _sc_scatter_body(src_hbm, dst_hbm, a_hbm, *scr):
    srcA, dstA, srcB, dstB, idxc, ones_v, semA, semB, semS = scr

    wid = lax.axis_index("s") * _NC + lax.axis_index("c")
    for v in range(_GE // _L):
        ones_v[pl.ds(v * _L, _L)] = jnp.full((_L,), 1.0, jnp.float32)

    start = wid * _STRIDE * _CH
    max_base = N_EDGES - _GE

    def issue_loads(g, sbuf, dbuf, sem):
        base = jnp.minimum(start + g * _GE, max_base)
        pltpu.async_copy(src_hbm.at[pl.ds(base, _GE)], sbuf, sem)
        pltpu.async_copy(dst_hbm.at[pl.ds(base, _GE)], dbuf, sem)

    def drain_loads(sbuf, dbuf, sem):
        pltpu.make_async_copy(src_hbm.at[pl.ds(0, _GE)], sbuf, sem).wait()
        pltpu.make_async_copy(dst_hbm.at[pl.ds(0, _GE)], dbuf, sem).wait()

    def process_group(sbuf, dbuf):
        for v in range(_GE // _L):
            sl = pl.ds(v * _L, _L)
            sv = sbuf[sl]
            dv = dbuf[sl]
            lo = jnp.minimum(sv, dv)
            hi = jnp.maximum(sv, dv)
            idxc[sl] = lo * N_PAD + hi
        return [pltpu.async_copy(ones_v, a_hbm.at[idxc], semS)]

    issue_loads(0, srcA, dstA, semA)
    issue_loads(1, srcB, dstB, semB)

    def body(g2, carry):
        ga = 2 * g2
        drain_loads(srcA, dstA, semA)
        cps = process_group(srcA, dstA)
        issue_loads(ga + 2, srcA, dstA, semA)
        for cp in cps:
            cp.wait()
        gb = ga + 1
        drain_loads(srcB, dstB, semB)
        cps = process_group(srcB, dstB)
        issue_loads(gb + 2, srcB, dstB, semB)
        for cp in cps:
            cp.wait()
        return carry

    lax.fori_loop(0, _NG // 2, body, 0)
    drain_loads(srcA, dstA, semA)
    cps = process_group(srcA, dstA)
    for cp in cps:
        cp.wait()
    drain_loads(srcB, dstB, semB)


@functools.cache
def _get_sc_scatter():
    scratch = [
        pltpu.VMEM((_GE,), jnp.int32),
        pltpu.VMEM((_GE,), jnp.int32),
        pltpu.VMEM((_GE,), jnp.int32),
        pltpu.VMEM((_GE,), jnp.int32),
        pltpu.VMEM((_GE,), jnp.int32),
        pltpu.VMEM((_GE,), jnp.float32),
        pltpu.SemaphoreType.DMA,
        pltpu.SemaphoreType.DMA,
        pltpu.SemaphoreType.DMA,
    ]
    return pl.kernel(
        _sc_scatter_body,
        out_type=(),
        mesh=plsc.VectorSubcoreMesh(core_axis_name="c", subcore_axis_name="s"),
        scratch_types=scratch,
    )


_BM = 1024
_NB = N_PAD // _BM
_NJ = _NB // 2 + 1


def _acc_body(u_blk, xaug_ref, acc_out, acc_ref, acc2t_ref):
    i = pl.program_id(0)
    jj = pl.program_id(1)
    j = lax.rem(i + jj, _NB)
    r = jnp.minimum(i, j)
    c = jnp.maximum(i, j)

    @pl.when((i == 0) & (jj == 0))
    def _init():
        acc_ref[...] = jnp.zeros_like(acc_ref)
        acc2t_ref[...] = jnp.zeros_like(acc2t_ref)

    @pl.when((jj < _NJ - 1) | (i < _NB // 2))
    def _accumulate():
        blk = u_blk[...].astype(jnp.bfloat16)
        x_c = xaug_ref[pl.ds(c * _BM, _BM), :]
        x_r = xaug_ref[pl.ds(r * _BM, _BM), :]
        acc_ref[pl.ds(r * _BM, _BM), :] += jnp.dot(
            blk, x_c, preferred_element_type=jnp.float32)
        ir = lax.broadcasted_iota(jnp.int32, (_BM, _BM), 0)
        ic = lax.broadcasted_iota(jnp.int32, (_BM, _BM), 1)
        tblk = jnp.where((r == c) & (ir == ic), jnp.bfloat16(0.0), blk)
        acc2t_ref[:, pl.ds(c * _BM, _BM)] += lax.dot_general(
            x_r, tblk, (((0,), (0,)), ((), ())),
            preferred_element_type=jnp.float32)

    @pl.when((i == _NB - 1) & (jj == _NJ - 1))
    def _flush():
        acc_out[...] = acc_ref[...] + acc2t_ref[...].T


_tc_acc = pl.pallas_call(
    _acc_body,
    grid=(_NB, _NJ),
    in_specs=[
        pl.BlockSpec((_BM, _BM),
                     lambda i, jj: (jnp.minimum(i, lax.rem(i + jj, _NB)),
                                    jnp.maximum(i, lax.rem(i + jj, _NB)))),
        pl.BlockSpec((N_PAD, 2 * FEAT), lambda i, jj: (0, 0)),
    ],
    out_specs=pl.BlockSpec((N_PAD, 2 * FEAT), lambda i, jj: (0, 0)),
    out_shape=jax.ShapeDtypeStruct((N_PAD, 2 * FEAT), jnp.float32),
    scratch_shapes=[pltpu.VMEM((N_PAD, 2 * FEAT), jnp.float32),
                    pltpu.VMEM((2 * FEAT, N_PAD), jnp.float32)],
    compiler_params=pltpu.CompilerParams(
        dimension_semantics=("arbitrary", "arbitrary")),
)


def _mlp_body(acc_blk, x_blk, w1_ref, b1_ref, w2_ref, b2_ref,
              w3_ref, b3_ref, upd_ref, tap_ref, ssum_ref, viol_ref):
    i = pl.program_id(0)

    x = x_blk[...]
    acc = acc_blk[...]
    nsum = acc[:, :FEAT]
    cnt = acc[:, FEAT:FEAT + 1]
    has_nb = cnt > 0.0
    nmean = nsum / jnp.maximum(cnt, 1.0)
    combined = jnp.concatenate([x, nmean], axis=1)
    h = jnp.maximum(
        jnp.dot(combined, w1_ref[...],
                preferred_element_type=jnp.float32) + b1_ref[...], 0.0)
    h = jnp.maximum(
        jnp.dot(h, w2_ref[...],
                preferred_element_type=jnp.float32) + b2_ref[...], 0.0)
    logits = jnp.sum(h * w3_ref[...], axis=1, keepdims=True) + b3_ref[...]
    score = jax.nn.sigmoid(logits)
    gain = jnp.where(has_nb, 0.05 * score, 0.0)
    upd_ref[...] = x + gain * jnp.tanh(x)
    tap = jnp.where(has_nb, score, 1.0)
    tap_ref[...] = tap
    part_sum = jnp.sum(tap).reshape(1, 1)
    part_viol = jnp.sum((tap < 0.7).astype(jnp.int32)).reshape(1, 1)

    @pl.when(i == 0)
    def _first():
        ssum_ref[...] = part_sum
        viol_ref[...] = part_viol

    @pl.when(i > 0)
    def _rest():
        ssum_ref[...] += part_sum
        viol_ref[...] += part_viol


_tc_mlp = pl.pallas_call(
    _mlp_body,
    grid=(_NB,),
    in_specs=[
        pl.BlockSpec((_BM, 2 * FEAT), lambda i: (i, 0)),
        pl.BlockSpec((_BM, FEAT), lambda i: (i, 0)),
        pl.BlockSpec((2 * FEAT, 64), lambda i: (0, 0)),
        pl.BlockSpec((1, 64), lambda i: (0, 0)),
        pl.BlockSpec((64, 32), lambda i: (0, 0)),
        pl.BlockSpec((1, 32), lambda i: (0, 0)),
        pl.BlockSpec((1, 32), lambda i: (0, 0)),
        pl.BlockSpec((1, 1), lambda i: (0, 0)),
    ],
    out_specs=[
        pl.BlockSpec((_BM, FEAT), lambda i: (i, 0)),
        pl.BlockSpec((_BM, 1), lambda i: (i, 0)),
        pl.BlockSpec((1, 1), lambda i: (0, 0)),
        pl.BlockSpec((1, 1), lambda i: (0, 0)),
    ],
    out_shape=[
        jax.ShapeDtypeStruct((N_PAD, FEAT), jnp.float32),
        jax.ShapeDtypeStruct((N_PAD, 1), jnp.float32),
        jax.ShapeDtypeStruct((1, 1), jnp.float32),
        jax.ShapeDtypeStruct((1, 1), jnp.int32),
    ],
    compiler_params=pltpu.CompilerParams(
        dimension_semantics=("arbitrary",)),
)


def kernel(node_features, edge_index, node_positions, node_radii,
           W1, b1, W2, b2, W3, b3):
    del node_positions, node_radii
    src = edge_index[0].astype(jnp.int32)
    dst = edge_index[1].astype(jnp.int32)

    a_ref = jax.new_ref(jnp.zeros((N_PAD * N_PAD,), jnp.float32))
    _get_sc_scatter()(src, dst, a_ref)
    U = a_ref[...].reshape(N_PAD, N_PAD)

    xaug = jnp.concatenate(
        [node_features,
         jnp.ones((N_NODES, 1), jnp.float32),
         jnp.zeros((N_NODES, FEAT - 1), jnp.float32)], axis=1
    ).astype(jnp.bfloat16)
    xaug_p = jnp.pad(xaug, ((0, N_PAD - N_NODES), (0, 0)))
    x_p = jnp.pad(node_features, ((0, N_PAD - N_NODES), (0, 0)))

    acc = _tc_acc(U, xaug_p)
    updated_p, tap_p, ssum, viol = _tc_mlp(
        acc, x_p,
        W1, b1.reshape(1, 64), W2, b2.reshape(1, 32),
        W3.reshape(1, 32), b3.reshape(1, 1))

    updated = updated_p[:N_NODES]
    tapering_scores = tap_p[:N_NODES, 0]
    avg_consistency = (ssum[0, 0] - np.float32(N_PAD - N_NODES)) / np.float32(N_NODES)
    num_violations = viol[0, 0]
    return updated, tapering_scores, avg_consistency, num_violations


# speedup vs baseline: 1.4330x; 1.0114x over previous
import functools

import jax
import jax.numpy as jnp
import numpy as np
from jax import lax
from jax.experimental import pallas as pl
from jax.experimental.pallas import tpu as pltpu
from jax.experimental.pallas import tpu_sc as plsc

N_NODES = 10000
N_PAD = 10240
N_EDGES = 320000
FEAT = 128

_NC, _NS, _L = 2, 16, 16
_NW = _NC * _NS
_CH = 128
_NCHUNKS = N_EDGES // _CH

_GCH = 16
_GE = _GCH * _CH
_NG = 5
_STRIDE = 79


def _sc_scatter_body(src_hbm, dst_hbm, a_hbm, *scr):
    srcA, dstA, srcB, dstB, idxc, ones_v, semA, semB, semS = scr

    wid = lax.axis_index("s") * _NC + lax.axis_index("c")
    for v in range(_GE // _L):
        ones_v[pl.ds(v * _L, _L)] = jnp.full((_L,), 1.0, jnp.float32)

    start = wid * _STRIDE * _CH
    max_base = N_EDGES - _GE

    def issue_loads(g, sbuf, dbuf, sem):
        base = jnp.minimum(start + g * _GE, max_base)
        pltpu.async_copy(src_hbm.at[pl.ds(base, _GE)], sbuf, sem)
        pltpu.async_copy(dst_hbm.at[pl.ds(base, _GE)], dbuf, sem)

    def drain_loads(sbuf, dbuf, sem):
        pltpu.make_async_copy(src_hbm.at[pl.ds(0, _GE)], sbuf, sem).wait()
        pltpu.make_async_copy(dst_hbm.at[pl.ds(0, _GE)], dbuf, sem).wait()

    def process_group(sbuf, dbuf):
        for v in range(_GE // _L):
            sl = pl.ds(v * _L, _L)
            sv = sbuf[sl]
            dv = dbuf[sl]
            lo = jnp.minimum(sv, dv)
            hi = jnp.maximum(sv, dv)
            idxc[sl] = lo * N_PAD + hi
        return [pltpu.async_copy(ones_v, a_hbm.at[idxc], semS)]

    issue_loads(0, srcA, dstA, semA)
    issue_loads(1, srcB, dstB, semB)

    def body(g2, carry):
        ga = 2 * g2
        drain_loads(srcA, dstA, semA)
        cps = process_group(srcA, dstA)
        issue_loads(ga + 2, srcA, dstA, semA)
        for cp in cps:
            cp.wait()
        gb = ga + 1
        drain_loads(srcB, dstB, semB)
        cps = process_group(srcB, dstB)
        issue_loads(gb + 2, srcB, dstB, semB)
        for cp in cps:
            cp.wait()
        return carry

    lax.fori_loop(0, _NG // 2, body, 0)
    drain_loads(srcA, dstA, semA)
    cps = process_group(srcA, dstA)
    for cp in cps:
        cp.wait()
    drain_loads(srcB, dstB, semB)


@functools.cache
def _get_sc_scatter():
    scratch = [
        pltpu.VMEM((_GE,), jnp.int32),
        pltpu.VMEM((_GE,), jnp.int32),
        pltpu.VMEM((_GE,), jnp.int32),
        pltpu.VMEM((_GE,), jnp.int32),
        pltpu.VMEM((_GE,), jnp.int32),
        pltpu.VMEM((_GE,), jnp.float32),
        pltpu.SemaphoreType.DMA,
        pltpu.SemaphoreType.DMA,
        pltpu.SemaphoreType.DMA,
    ]
    return pl.kernel(
        _sc_scatter_body,
        out_type=(),
        mesh=plsc.VectorSubcoreMesh(core_axis_name="c", subcore_axis_name="s"),
        scratch_types=scratch,
    )


_BM = 1024
_NB = N_PAD // _BM
_NJ = _NB // 2 + 1


def _acc_body(u_blk, xaug_ref, x_ref, w1_ref, b1_ref, w2_ref, b2_ref,
              w3_ref, b3_ref, upd_ref, tap_ref, ssum_ref, viol_ref,
              acc_ref, acc2t_ref):
    i = pl.program_id(0)
    jj = pl.program_id(1)
    j = lax.rem(i + jj, _NB)
    r = jnp.minimum(i, j)
    c = jnp.maximum(i, j)

    @pl.when((i == 0) & (jj == 0))
    def _init():
        acc_ref[...] = jnp.zeros_like(acc_ref)
        acc2t_ref[...] = jnp.zeros_like(acc2t_ref)

    @pl.when((jj < _NJ - 1) | (i < _NB // 2))
    def _accumulate():
        blk = u_blk[...].astype(jnp.bfloat16)
        x_c = xaug_ref[pl.ds(c * _BM, _BM), :]
        x_r = xaug_ref[pl.ds(r * _BM, _BM), :]
        acc_ref[pl.ds(r * _BM, _BM), :] += jnp.dot(
            blk, x_c, preferred_element_type=jnp.float32)
        ir = lax.broadcasted_iota(jnp.int32, (_BM, _BM), 0)
        ic = lax.broadcasted_iota(jnp.int32, (_BM, _BM), 1)
        tblk = jnp.where((r == c) & (ir == ic), jnp.bfloat16(0.0), blk)
        acc2t_ref[:, pl.ds(c * _BM, _BM)] += lax.dot_general(
            x_r, tblk, (((0,), (0,)), ((), ())),
            preferred_element_type=jnp.float32)

    @pl.when((i == _NB - 1) & (jj == _NJ - 1))
    def _flush():
        tot_sum = jnp.zeros((1, 1), jnp.float32)
        tot_viol = jnp.zeros((1, 1), jnp.int32)
        for b in range(_NB):
            ds = pl.ds(b * _BM, _BM)
            acc = acc_ref[ds, :] + acc2t_ref[:, ds].T
            x = x_ref[ds, :]
            nsum = acc[:, :FEAT]
            cnt = acc[:, FEAT:FEAT + 1]
            has_nb = cnt > 0.0
            nmean = nsum / jnp.maximum(cnt, 1.0)
            combined = jnp.concatenate([x, nmean], axis=1)
            h = jnp.maximum(
                jnp.dot(combined, w1_ref[...],
                        preferred_element_type=jnp.float32) + b1_ref[...], 0.0)
            h = jnp.maximum(
                jnp.dot(h, w2_ref[...],
                        preferred_element_type=jnp.float32) + b2_ref[...], 0.0)
            logits = (jnp.sum(h * w3_ref[...], axis=1, keepdims=True)
                      + b3_ref[...])
            score = jax.nn.sigmoid(logits)
            gain = jnp.where(has_nb, 0.05 * score, 0.0)
            upd_ref[ds, :] = x + gain * jnp.tanh(x)
            tap = jnp.where(has_nb, score, 1.0)
            tap_ref[ds, :] = tap
            tot_sum = tot_sum + jnp.sum(tap).reshape(1, 1)
            tot_viol = tot_viol + jnp.sum((tap < 0.7).astype(jnp.int32)
                                          ).reshape(1, 1)
        ssum_ref[...] = tot_sum
        viol_ref[...] = tot_viol


_tc_acc = pl.pallas_call(
    _acc_body,
    grid=(_NB, _NJ),
    in_specs=[
        pl.BlockSpec((_BM, _BM),
                     lambda i, jj: (jnp.minimum(i, lax.rem(i + jj, _NB)),
                                    jnp.maximum(i, lax.rem(i + jj, _NB)))),
        pl.BlockSpec((N_PAD, 2 * FEAT), lambda i, jj: (0, 0)),
        pl.BlockSpec((N_PAD, FEAT), lambda i, jj: (0, 0)),
        pl.BlockSpec((2 * FEAT, 64), lambda i, jj: (0, 0)),
        pl.BlockSpec((1, 64), lambda i, jj: (0, 0)),
        pl.BlockSpec((64, 32), lambda i, jj: (0, 0)),
        pl.BlockSpec((1, 32), lambda i, jj: (0, 0)),
        pl.BlockSpec((1, 32), lambda i, jj: (0, 0)),
        pl.BlockSpec((1, 1), lambda i, jj: (0, 0)),
    ],
    out_specs=[
        pl.BlockSpec((N_PAD, FEAT), lambda i, jj: (0, 0)),
        pl.BlockSpec((N_PAD, 1), lambda i, jj: (0, 0)),
        pl.BlockSpec((1, 1), lambda i, jj: (0, 0)),
        pl.BlockSpec((1, 1), lambda i, jj: (0, 0)),
    ],
    out_shape=[
        jax.ShapeDtypeStruct((N_PAD, FEAT), jnp.float32),
        jax.ShapeDtypeStruct((N_PAD, 1), jnp.float32),
        jax.ShapeDtypeStruct((1, 1), jnp.float32),
        jax.ShapeDtypeStruct((1, 1), jnp.int32),
    ],
    scratch_shapes=[pltpu.VMEM((N_PAD, 2 * FEAT), jnp.float32),
                    pltpu.VMEM((2 * FEAT, N_PAD), jnp.float32)],
    compiler_params=pltpu.CompilerParams(
        dimension_semantics=("arbitrary", "arbitrary")),
)


def kernel(node_features, edge_index, node_positions, node_radii,
           W1, b1, W2, b2, W3, b3):
    del node_positions, node_radii
    src = edge_index[0].astype(jnp.int32)
    dst = edge_index[1].astype(jnp.int32)

    a_ref = jax.new_ref(jnp.zeros((N_PAD * N_PAD,), jnp.float32))
    _get_sc_scatter()(src, dst, a_ref)
    U = a_ref[...].reshape(N_PAD, N_PAD)

    xaug = jnp.concatenate(
        [node_features,
         jnp.ones((N_NODES, 1), jnp.float32),
         jnp.zeros((N_NODES, FEAT - 1), jnp.float32)], axis=1
    ).astype(jnp.bfloat16)
    xaug_p = jnp.pad(xaug, ((0, N_PAD - N_NODES), (0, 0)))
    x_p = jnp.pad(node_features, ((0, N_PAD - N_NODES), (0, 0)))

    updated_p, tap_p, ssum, viol = _tc_acc(
        U, xaug_p, x_p,
        W1, b1.reshape(1, 64), W2, b2.reshape(1, 32),
        W3.reshape(1, 32), b3.reshape(1, 1))

    updated = updated_p[:N_NODES]
    tapering_scores = tap_p[:N_NODES, 0]
    avg_consistency = (ssum[0, 0] - np.float32(N_PAD - N_NODES)) / np.float32(N_NODES)
    num_violations = viol[0, 0]
    return updated, tapering_scores, avg_consistency, num_violations


# speedup vs baseline: 1.8177x; 1.2685x over previous
import functools

import jax
import jax.numpy as jnp
import numpy as np
from jax import lax
from jax.experimental import pallas as pl
from jax.experimental.pallas import tpu as pltpu
from jax.experimental.pallas import tpu_sc as plsc

N_NODES = 10000
N_PAD = 10240
N_EDGES = 320000
FEAT = 128

_NC, _NS, _L = 2, 16, 16
_NW = _NC * _NS
_CH = 128
_NCHUNKS = N_EDGES // _CH

_GCH = 16
_GE = _GCH * _CH
_NG = 5
_STRIDE = 79

_BMB = 1024
_NBB = N_PAD // _BMB
_NSLOT = _NBB * (_NBB + 1) // 2


def _sc_scatter_body(src_hbm, dst_hbm, a_hbm, *scr):
    srcA, dstA, srcB, dstB, idxc, ones_v, semA, semB, semS = scr

    wid = lax.axis_index("s") * _NC + lax.axis_index("c")
    for v in range(_GE // _L):
        ones_v[pl.ds(v * _L, _L)] = jnp.full((_L,), 1.0, jnp.float32)

    start = wid * _STRIDE * _CH
    max_base = N_EDGES - _GE

    def issue_loads(g, sbuf, dbuf, sem):
        base = jnp.minimum(start + g * _GE, max_base)
        pltpu.async_copy(src_hbm.at[pl.ds(base, _GE)], sbuf, sem)
        pltpu.async_copy(dst_hbm.at[pl.ds(base, _GE)], dbuf, sem)

    def drain_loads(sbuf, dbuf, sem):
        pltpu.make_async_copy(src_hbm.at[pl.ds(0, _GE)], sbuf, sem).wait()
        pltpu.make_async_copy(dst_hbm.at[pl.ds(0, _GE)], dbuf, sem).wait()

    def process_group(sbuf, dbuf):
        for v in range(_GE // _L):
            sl = pl.ds(v * _L, _L)
            sv = sbuf[sl]
            dv = dbuf[sl]
            lo = jnp.minimum(sv, dv)
            hi = jnp.maximum(sv, dv)
            bi = lax.shift_right_logical(lo, 10)
            bj = lax.shift_right_logical(hi, 10)
            slot = (bi * _NBB - lax.shift_right_logical(bi * (bi - 1), 1)
                    + (bj - bi))
            lr = lax.bitwise_and(lo, _BMB - 1)
            lc = lax.bitwise_and(hi, _BMB - 1)
            idxc[sl] = (lax.shift_left(slot, 20) + lax.shift_left(lr, 10)
                        + lc)
        return [pltpu.async_copy(ones_v, a_hbm.at[idxc], semS)]

    issue_loads(0, srcA, dstA, semA)
    issue_loads(1, srcB, dstB, semB)

    def body(g2, carry):
        ga = 2 * g2
        drain_loads(srcA, dstA, semA)
        cps = process_group(srcA, dstA)
        issue_loads(ga + 2, srcA, dstA, semA)
        for cp in cps:
            cp.wait()
        gb = ga + 1
        drain_loads(srcB, dstB, semB)
        cps = process_group(srcB, dstB)
        issue_loads(gb + 2, srcB, dstB, semB)
        for cp in cps:
            cp.wait()
        return carry

    lax.fori_loop(0, _NG // 2, body, 0)
    drain_loads(srcA, dstA, semA)
    cps = process_group(srcA, dstA)
    for cp in cps:
        cp.wait()
    drain_loads(srcB, dstB, semB)


@functools.cache
def _get_sc_scatter():
    scratch = [
        pltpu.VMEM((_GE,), jnp.int32),
        pltpu.VMEM((_GE,), jnp.int32),
        pltpu.VMEM((_GE,), jnp.int32),
        pltpu.VMEM((_GE,), jnp.int32),
        pltpu.VMEM((_GE,), jnp.int32),
        pltpu.VMEM((_GE,), jnp.float32),
        pltpu.SemaphoreType.DMA,
        pltpu.SemaphoreType.DMA,
        pltpu.SemaphoreType.DMA,
    ]
    return pl.kernel(
        _sc_scatter_body,
        out_type=(),
        mesh=plsc.VectorSubcoreMesh(core_axis_name="c", subcore_axis_name="s"),
        scratch_types=scratch,
    )


_BM = 1024
_NB = N_PAD // _BM
_NJ = _NB // 2 + 1


def _acc_body(u_blk, xaug_ref, x_ref, w1_ref, b1_ref, w2_ref, b2_ref,
              w3_ref, b3_ref, upd_ref, tap_ref, ssum_ref, viol_ref,
              acc_ref, acc2t_ref):
    i = pl.program_id(0)
    jj = pl.program_id(1)
    j = lax.rem(i + jj, _NB)
    r = jnp.minimum(i, j)
    c = jnp.maximum(i, j)

    @pl.when((i == 0) & (jj == 0))
    def _init():
        acc_ref[...] = jnp.zeros_like(acc_ref)
        acc2t_ref[...] = jnp.zeros_like(acc2t_ref)

    @pl.when((jj < _NJ - 1) | (i < _NB // 2))
    def _accumulate():
        blk = u_blk[...].astype(jnp.bfloat16)
        x_c = xaug_ref[pl.ds(c * _BM, _BM), :]
        x_r = xaug_ref[pl.ds(r * _BM, _BM), :]
        acc_ref[pl.ds(r * _BM, _BM), :] += jnp.dot(
            blk, x_c, preferred_element_type=jnp.float32)
        ir = lax.broadcasted_iota(jnp.int32, (_BM, _BM), 0)
        ic = lax.broadcasted_iota(jnp.int32, (_BM, _BM), 1)
        tblk = jnp.where((r == c) & (ir == ic), jnp.bfloat16(0.0), blk)
        acc2t_ref[:, pl.ds(c * _BM, _BM)] += lax.dot_general(
            x_r, tblk, (((0,), (0,)), ((), ())),
            preferred_element_type=jnp.float32)

    @pl.when((i == _NB - 1) & (jj == _NJ - 1))
    def _flush():
        tot_sum = jnp.zeros((1, 1), jnp.float32)
        tot_viol = jnp.zeros((1, 1), jnp.int32)
        for b in range(_NB):
            ds = pl.ds(b * _BM, _BM)
            acc = acc_ref[ds, :] + acc2t_ref[:, ds].T
            x = x_ref[ds, :]
            nsum = acc[:, :FEAT]
            cnt = acc[:, FEAT:FEAT + 1]
            has_nb = cnt > 0.0
            nmean = nsum / jnp.maximum(cnt, 1.0)
            combined = jnp.concatenate([x, nmean], axis=1)
            h = jnp.maximum(
                jnp.dot(combined, w1_ref[...],
                        preferred_element_type=jnp.float32) + b1_ref[...], 0.0)
            h = jnp.maximum(
                jnp.dot(h, w2_ref[...],
                        preferred_element_type=jnp.float32) + b2_ref[...], 0.0)
            logits = (jnp.sum(h * w3_ref[...], axis=1, keepdims=True)
                      + b3_ref[...])
            score = jax.nn.sigmoid(logits)
            gain = jnp.where(has_nb, 0.05 * score, 0.0)
            upd_ref[ds, :] = x + gain * jnp.tanh(x)
            tap = jnp.where(has_nb, score, 1.0)
            tap_ref[ds, :] = tap
            tot_sum = tot_sum + jnp.sum(tap).reshape(1, 1)
            tot_viol = tot_viol + jnp.sum((tap < 0.7).astype(jnp.int32)
                                          ).reshape(1, 1)
        ssum_ref[...] = tot_sum
        viol_ref[...] = tot_viol


_tc_acc = pl.pallas_call(
    _acc_body,
    grid=(_NB, _NJ),
    in_specs=[
        pl.BlockSpec((_BM, _BM),
                     lambda i, jj: (
                         (lambda r, c: r * _NB
                          - lax.shift_right_logical(r * (r - 1), 1)
                          + (c - r))(
                              jnp.minimum(i, lax.rem(i + jj, _NB)),
                              jnp.maximum(i, lax.rem(i + jj, _NB))),
                         0)),
        pl.BlockSpec((N_PAD, 2 * FEAT), lambda i, jj: (0, 0)),
        pl.BlockSpec((N_PAD, FEAT), lambda i, jj: (0, 0)),
        pl.BlockSpec((2 * FEAT, 64), lambda i, jj: (0, 0)),
        pl.BlockSpec((1, 64), lambda i, jj: (0, 0)),
        pl.BlockSpec((64, 32), lambda i, jj: (0, 0)),
        pl.BlockSpec((1, 32), lambda i, jj: (0, 0)),
        pl.BlockSpec((1, 32), lambda i, jj: (0, 0)),
        pl.BlockSpec((1, 1), lambda i, jj: (0, 0)),
    ],
    out_specs=[
        pl.BlockSpec((N_PAD, FEAT), lambda i, jj: (0, 0)),
        pl.BlockSpec((N_PAD, 1), lambda i, jj: (0, 0)),
        pl.BlockSpec((1, 1), lambda i, jj: (0, 0)),
        pl.BlockSpec((1, 1), lambda i, jj: (0, 0)),
    ],
    out_shape=[
        jax.ShapeDtypeStruct((N_PAD, FEAT), jnp.float32),
        jax.ShapeDtypeStruct((N_PAD, 1), jnp.float32),
        jax.ShapeDtypeStruct((1, 1), jnp.float32),
        jax.ShapeDtypeStruct((1, 1), jnp.int32),
    ],
    scratch_shapes=[pltpu.VMEM((N_PAD, 2 * FEAT), jnp.float32),
                    pltpu.VMEM((2 * FEAT, N_PAD), jnp.float32)],
    compiler_params=pltpu.CompilerParams(
        dimension_semantics=("arbitrary", "arbitrary")),
)


def kernel(node_features, edge_index, node_positions, node_radii,
           W1, b1, W2, b2, W3, b3):
    del node_positions, node_radii
    src = edge_index[0].astype(jnp.int32)
    dst = edge_index[1].astype(jnp.int32)

    a_ref = jax.new_ref(jnp.zeros((_NSLOT * _BMB * _BMB,), jnp.float32))
    _get_sc_scatter()(src, dst, a_ref)
    U = a_ref[...].reshape(_NSLOT * _BMB, _BMB)

    xaug = jnp.concatenate(
        [node_features,
         jnp.ones((N_NODES, 1), jnp.float32),
         jnp.zeros((N_NODES, FEAT - 1), jnp.float32)], axis=1
    ).astype(jnp.bfloat16)
    xaug_p = jnp.pad(xaug, ((0, N_PAD - N_NODES), (0, 0)))
    x_p = jnp.pad(node_features, ((0, N_PAD - N_NODES), (0, 0)))

    updated_p, tap_p, ssum, viol = _tc_acc(
        U, xaug_p, x_p,
        W1, b1.reshape(1, 64), W2, b2.reshape(1, 32),
        W3.reshape(1, 32), b3.reshape(1, 1))

    updated = updated_p[:N_NODES]
    tapering_scores = tap_p[:N_NODES, 0]
    avg_consistency = (ssum[0, 0] - np.float32(N_PAD - N_NODES)) / np.float32(N_NODES)
    num_violations = viol[0, 0]
    return updated, tapering_scores, avg_consistency, num_violations
